# baseline (speedup 1.0000x reference)
import functools

import jax
import jax.numpy as jnp
from jax import lax
from jax.experimental import pallas as pl
from jax.experimental.pallas import tpu as pltpu
from jax.experimental.pallas import tpu_sc as plsc

VOCAB = 100
HIDDEN = 2048
HALF = HIDDEN // 2
PRE = 16
BATCH = 4
SEQ = 2048
ROWS_PER_BATCH = PRE + SEQ
TOTAL_ROWS = BATCH * ROWS_PER_BATCH
TOKENS = BATCH * SEQ

NC = 2
NS = 16
TOK_PER_T = TOKENS // NS
CK = 8
NCHUNK = TOK_PER_T // CK
LANES = 16
VPT = HALF // LANES

_mesh = plsc.VectorSubcoreMesh(core_axis_name="c", subcore_axis_name="s")


@functools.partial(
    pl.kernel,
    mesh=_mesh,
    out_type=jax.ShapeDtypeStruct((TOTAL_ROWS, HIDDEN), jnp.float32),
    scratch_types=[
        pltpu.SMEM((TOK_PER_T,), jnp.int32),
        pltpu.VMEM_SHARED((TOKENS,), jnp.int32),
        pltpu.VMEM((VOCAB, HALF), jnp.float32),
        pltpu.VMEM((CK, HALF), jnp.float32),
        pltpu.VMEM((CK, HALF), jnp.float32),
        pltpu.SemaphoreType.DMA,
        pltpu.SemaphoreType.DMA,
        pltpu.SemaphoreType.DMA,
    ],
)
def _embed_sc(ids_hbm, tab0_hbm, tab1_hbm, pr0_hbm, pr1_hbm, out_hbm,
              ids_s, ids_sh, table_v, stage0, stage1,
              sem0, sem1, psem):
    c = lax.axis_index("c")
    s = lax.axis_index("s")
    wid = s * NC + c
    cofs = c * HALF
    tok_base = s * TOK_PER_T
    row_base = (s // 4) * ROWS_PER_BATCH + PRE + (s % 4) * TOK_PER_T

    @pl.when(c == 0)
    def _():
        pltpu.sync_copy(tab0_hbm, table_v)

    @pl.when(c == 1)
    def _():
        pltpu.sync_copy(tab1_hbm, table_v)

    pltpu.sync_copy(ids_hbm.at[pl.ds(tok_base, TOK_PER_T)],
                    ids_sh.at[pl.ds(tok_base, TOK_PER_T)])
    pltpu.sync_copy(ids_sh.at[pl.ds(tok_base, TOK_PER_T)], ids_s)

    pb = wid // 4
    ph = (wid % 4) // 2
    pr = (wid % 2) * CK

    @pl.when(wid < 4 * BATCH)
    def _():
        @pl.when(ph == 0)
        def _():
            pltpu.async_copy(pr0_hbm.at[pl.ds(pr, CK)], stage0, psem).wait()

        @pl.when(ph == 1)
        def _():
            pltpu.async_copy(pr1_hbm.at[pl.ds(pr, CK)], stage0, psem).wait()

        pltpu.async_copy(
            stage0,
            out_hbm.at[pl.ds(pb * ROWS_PER_BATCH + pr, CK),
                       pl.ds(ph * HALF, HALF)], psem).wait()

    stages = (stage0, stage1)
    sems = (sem0, sem1)

    def _fill(stage, chunk):
        @plsc.parallel_loop(0, CK, 1, unroll=CK)
        def _(t):
            idx = ids_s[chunk * CK + t]

            @plsc.parallel_loop(0, VPT, 1, unroll=4)
            def _(k):
                stage[t, pl.ds(k * LANES, LANES)] = (
                    table_v[idx, pl.ds(k * LANES, LANES)])

    def _write(stage, sem, chunk):
        return pltpu.async_copy(
            stage,
            out_hbm.at[pl.ds(row_base + chunk * CK, CK), pl.ds(cofs, HALF)],
            sem)

    _fill(stage0, 0)
    _write(stage0, sem0, 0)
    _fill(stage1, 1)
    _write(stage1, sem1, 1)

    def _body(i, carry):
        for b in range(2):
            chunk = i * 2 + b
            pltpu.make_async_copy(
                stages[b],
                out_hbm.at[pl.ds(row_base, CK), pl.ds(cofs, HALF)],
                sems[b]).wait()
            _fill(stages[b], chunk)
            _write(stages[b], sems[b], chunk)
        return carry

    lax.fori_loop(1, NCHUNK // 2, _body, 0)

    for b in range(2):
        pltpu.make_async_copy(
            stages[b],
            out_hbm.at[pl.ds(row_base, CK), pl.ds(cofs, HALF)],
            sems[b]).wait()


def kernel(input_ids, word_emb, prompt_emb):
    ids = jnp.asarray(input_ids, jnp.int32).reshape(-1)
    tab0 = word_emb[:, :HALF]
    tab1 = word_emb[:, HALF:]
    pr0 = prompt_emb[:, :HALF]
    pr1 = prompt_emb[:, HALF:]
    out = _embed_sc(ids, tab0, tab1, pr0, pr1)
    return out.reshape(BATCH, ROWS_PER_BATCH, HIDDEN)


# speedup vs baseline: 1.2716x; 1.2716x over previous
import functools

import jax
import jax.numpy as jnp
from jax import lax
from jax.experimental import pallas as pl
from jax.experimental.pallas import tpu as pltpu
from jax.experimental.pallas import tpu_sc as plsc

VOCAB = 100
HIDDEN = 2048
HALF = HIDDEN // 2
PRE = 16
BATCH = 4
SEQ = 2048
ROWS_PER_BATCH = PRE + SEQ
TOTAL_ROWS = BATCH * ROWS_PER_BATCH
TOKENS = BATCH * SEQ

NC = 2
NS = 16
TOK_PER_T = TOKENS // NS
CK = 8
NCHUNK = TOK_PER_T // CK
LANES = 16
VPT = HALF // LANES

_mesh = plsc.VectorSubcoreMesh(core_axis_name="c", subcore_axis_name="s")


@functools.partial(
    pl.kernel,
    mesh=_mesh,
    out_type=jax.ShapeDtypeStruct((TOTAL_ROWS, HIDDEN), jnp.float32),
    scratch_types=[
        pltpu.SMEM((TOK_PER_T,), jnp.int32),
        pltpu.VMEM_SHARED((TOKENS,), jnp.int32),
        pltpu.VMEM((VOCAB, HALF), jnp.float32),
        pltpu.VMEM((CK, HALF), jnp.float32),
        pltpu.VMEM((CK, HALF), jnp.float32),
        pltpu.SemaphoreType.DMA,
        pltpu.SemaphoreType.DMA,
        pltpu.SemaphoreType.DMA,
    ],
)
def _embed_sc(ids_hbm, tab0_hbm, tab1_hbm, pr0_hbm, pr1_hbm, out_hbm,
              ids_s, ids_sh, table_v, stage0, stage1,
              sem0, sem1, psem):
    c = lax.axis_index("c")
    s = lax.axis_index("s")
    wid = s * NC + c
    cofs = c * HALF
    tok_base = s * TOK_PER_T
    row_base = (s // 4) * ROWS_PER_BATCH + PRE + (s % 4) * TOK_PER_T

    @pl.when(c == 0)
    def _():
        pltpu.sync_copy(tab0_hbm, table_v)

    @pl.when(c == 1)
    def _():
        pltpu.sync_copy(tab1_hbm, table_v)

    pltpu.sync_copy(ids_hbm.at[pl.ds(tok_base, TOK_PER_T)],
                    ids_sh.at[pl.ds(tok_base, TOK_PER_T)])
    pltpu.sync_copy(ids_sh.at[pl.ds(tok_base, TOK_PER_T)], ids_s)

    pb = wid // 4
    ph = (wid % 4) // 2
    pr = (wid % 2) * CK

    @pl.when(wid < 4 * BATCH)
    def _():
        @pl.when(ph == 0)
        def _():
            pltpu.async_copy(pr0_hbm.at[pl.ds(pr, CK)], stage0, psem).wait()

        @pl.when(ph == 1)
        def _():
            pltpu.async_copy(pr1_hbm.at[pl.ds(pr, CK)], stage0, psem).wait()

        pltpu.async_copy(
            stage0,
            out_hbm.at[pl.ds(pb * ROWS_PER_BATCH + pr, CK),
                       pl.ds(ph * HALF, HALF)], psem).wait()

    stages = (stage0, stage1)
    sems = (sem0, sem1)

    def _fill(stage, chunk):
        @plsc.parallel_loop(0, CK, 1, unroll=4)
        def _(t):
            idx = ids_s[chunk * CK + t]

            @plsc.parallel_loop(0, VPT, 1, unroll=8)
            def _(k):
                stage[t, pl.ds(k * LANES, LANES)] = (
                    table_v[idx, pl.ds(k * LANES, LANES)])

    def _write(stage, sem, chunk):
        return pltpu.async_copy(
            stage,
            out_hbm.at[pl.ds(row_base + chunk * CK, CK), pl.ds(cofs, HALF)],
            sem)

    _fill(stage0, 0)
    _write(stage0, sem0, 0)
    _fill(stage1, 1)
    _write(stage1, sem1, 1)

    def _body(i, carry):
        for b in range(2):
            chunk = i * 2 + b
            pltpu.make_async_copy(
                stages[b],
                out_hbm.at[pl.ds(row_base, CK), pl.ds(cofs, HALF)],
                sems[b]).wait()
            _fill(stages[b], chunk)
            _write(stages[b], sems[b], chunk)
        return carry

    lax.fori_loop(1, NCHUNK // 2, _body, 0)

    for b in range(2):
        pltpu.make_async_copy(
            stages[b],
            out_hbm.at[pl.ds(row_base, CK), pl.ds(cofs, HALF)],
            sems[b]).wait()


def kernel(input_ids, word_emb, prompt_emb):
    ids = jnp.asarray(input_ids, jnp.int32).reshape(-1)
    tab0 = word_emb[:, :HALF]
    tab1 = word_emb[:, HALF:]
    pr0 = prompt_emb[:, :HALF]
    pr1 = prompt_emb[:, HALF:]
    out = _embed_sc(ids, tab0, tab1, pr0, pr1)
    return out.reshape(BATCH, ROWS_PER_BATCH, HIDDEN)
